# baseline (speedup 1.0000x reference)
import functools
import jax
import jax.numpy as jnp
from jax import lax
from jax.experimental import pallas as pl
from jax.experimental.pallas import tpu as pltpu
from jax.experimental.pallas import tpu_sc as plsc


def kernel(context):
    B, K, S, L, D = context.shape
    info = plsc.get_sparse_core_info()
    NC, NS = info.num_cores, info.num_subcores
    NW = NC * NS
    rows_per_w = B // NW

    mesh = plsc.VectorSubcoreMesh(core_axis_name="c", subcore_axis_name="s")

    @functools.partial(
        pl.kernel,
        mesh=mesh,
        out_type=jax.ShapeDtypeStruct((B, D), jnp.float32),
        scratch_types=[
            pltpu.VMEM((K, S, L, D), jnp.float32),
        ],
    )
    def probe(x_hbm, out_hbm, buf):
        wid = lax.axis_index("s") * NC + lax.axis_index("c")
        base = wid * rows_per_w

        def body(i, _):
            pltpu.sync_copy(x_hbm.at[base + i], buf)
            return 0

        lax.fori_loop(0, rows_per_w, body, 0)
        pltpu.sync_copy(
            buf.at[0, 0, pl.ds(0, rows_per_w), :],
            out_hbm.at[pl.ds(base, rows_per_w)],
        )

    return probe(context)


# speedup vs baseline: 1.0701x; 1.0701x over previous
import jax
import jax.numpy as jnp
from jax.experimental import pallas as pl
from jax.experimental.pallas import tpu as pltpu


def _body(x_ref, o_ref):
    bb, k, _, l, d = x_ref.shape
    ctx = x_ref[:, :, 0, :, :].reshape(bb * k, l, d)
    ent = x_ref[:, :, 1, :, :].reshape(bb * k, l, d)
    scores = jax.lax.dot_general(
        ctx, ent,
        dimension_numbers=(((2,), (2,)), ((0,), (0,))),
        preferred_element_type=jnp.float32,
    )
    o_ref[...] = jnp.sum(jnp.max(scores, axis=2), axis=1).reshape(bb, k)


def kernel(context):
    B, K, S, L, D = context.shape
    BLK_B = 32
    out = pl.pallas_call(
        _body,
        grid=(B // BLK_B,),
        in_specs=[pl.BlockSpec((BLK_B, K, S, L, D), lambda i: (i, 0, 0, 0, 0))],
        out_specs=pl.BlockSpec((BLK_B, K), lambda i: (i, 0)),
        out_shape=jax.ShapeDtypeStruct((B, K), jnp.float32),
    )(context)
    return out
